# Initial kernel scaffold; baseline (speedup 1.0000x reference)
import functools

import jax
import jax.numpy as jnp
from jax import lax
from jax.experimental import pallas as pl
from jax.experimental.pallas import tpu as pltpu
from jax.experimental.pallas import tpu_sc as plsc

N = 10000
E = 160000
D = 256
DH = 128
NS = 16
C = 80
EPW = E // NS
NCH = EPW // C
RPW = N // NS


def _sc_segment_sum(src3, dst3, x0, x1, z):
    mesh = plsc.VectorSubcoreMesh(core_axis_name="c", subcore_axis_name="s")

    @functools.partial(
        pl.kernel,
        out_type=jax.ShapeDtypeStruct((2, N, DH), jnp.float32),
        mesh=mesh,
        scratch_types=[
            pltpu.VMEM((NCH, C), jnp.int32),
            pltpu.VMEM((NCH, C), jnp.int32),
            pltpu.VMEM((C, DH), jnp.float32),
            pltpu.VMEM_SHARED((N, DH), jnp.float32),
            pltpu.SemaphoreType.DMA,
        ],
    )
    def sc_seg(src_hbm, dst_hbm, x0_hbm, x1_hbm, z_hbm, s_hbm,
               srcv, dstv, rows, acc, sem):
        c = lax.axis_index("c")
        s = lax.axis_index("s")
        base = s * RPW

        pltpu.async_copy(z_hbm.at[pl.ds(base, RPW)],
                         acc.at[pl.ds(base, RPW)], sem).wait()
        pltpu.async_copy(src_hbm.at[s], srcv, sem).wait()
        pltpu.async_copy(dst_hbm.at[s], dstv, sem).wait()
        plsc.subcore_barrier()

        def run(x_hbm):
            @pl.loop(0, NCH)
            def _(j):
                pltpu.async_copy(x_hbm.at[srcv.at[j]], rows, sem).wait()
                pltpu.sync_copy(rows, acc.at[dstv.at[j]], add=True)

        @pl.when(c == 0)
        def _():
            run(x0_hbm)

        @pl.when(c == 1)
        def _():
            run(x1_hbm)

        plsc.subcore_barrier()
        pltpu.async_copy(acc.at[pl.ds(base, RPW)],
                         s_hbm.at[c, pl.ds(base, RPW)], sem).wait()

    return sc_seg(src3, dst3, x0, x1, z)


BLK = 400


def _tc_matmul_body(s_ref, wt_ref, o_ref):
    a = jnp.dot(s_ref[0], wt_ref[:DH, :],
                preferred_element_type=jnp.float32,
                precision=lax.Precision.HIGHEST)
    b = jnp.dot(s_ref[1], wt_ref[DH:, :],
                preferred_element_type=jnp.float32,
                precision=lax.Precision.HIGHEST)
    o_ref[...] = a + b


_tc_matmul = functools.partial(
    pl.pallas_call,
    out_shape=jax.ShapeDtypeStruct((N, D), jnp.float32),
    grid=(N // BLK,),
    in_specs=[
        pl.BlockSpec((2, BLK, DH), lambda i: (0, i, 0)),
        pl.BlockSpec((D, D), lambda i: (0, 0)),
    ],
    out_specs=pl.BlockSpec((BLK, D), lambda i: (i, 0)),
)(_tc_matmul_body)


@jax.jit
def kernel(edge_index, X, W):
    src3 = edge_index[0].reshape(NS, NCH, C)
    dst3 = edge_index[1].reshape(NS, NCH, C)
    x0 = X[:, :DH]
    x1 = X[:, DH:]
    z = jnp.zeros((N, DH), jnp.float32)
    s2 = _sc_segment_sum(src3, dst3, x0, x1, z)
    return _tc_matmul(s2, W.T)


# speedup vs baseline: 4.9130x; 4.9130x over previous
import functools

import jax
import jax.numpy as jnp
from jax import lax
from jax.experimental import pallas as pl
from jax.experimental.pallas import tpu as pltpu
from jax.experimental.pallas import tpu_sc as plsc

N = 10000
E = 160000
D = 256
DH = 128
NS = 16
C = 80
EPW = E // NS
NCH = EPW // C
NP = 10240
RPW = NP // NS


def _sc_segment_sum(src3, dst3, x0, x1, z):
    mesh = plsc.VectorSubcoreMesh(core_axis_name="c", subcore_axis_name="s")

    @functools.partial(
        pl.kernel,
        out_type=jax.ShapeDtypeStruct((2, NP, DH), jnp.float32),
        mesh=mesh,
        scratch_types=[
            pltpu.VMEM((NCH, C), jnp.int32),
            pltpu.VMEM((NCH, C), jnp.int32),
            pltpu.VMEM((C, DH), jnp.float32),
            pltpu.VMEM_SHARED((NP, DH), jnp.float32),
            pltpu.SemaphoreType.DMA,
        ],
    )
    def sc_seg(src_hbm, dst_hbm, x0_hbm, x1_hbm, z_hbm, s_hbm,
               srcv, dstv, rows, acc, sem):
        c = lax.axis_index("c")
        s = lax.axis_index("s")
        base = s * RPW

        pltpu.async_copy(z_hbm.at[pl.ds(base, RPW)],
                         acc.at[pl.ds(base, RPW)], sem).wait()
        pltpu.async_copy(src_hbm.at[s], srcv, sem).wait()
        pltpu.async_copy(dst_hbm.at[s], dstv, sem).wait()
        plsc.subcore_barrier()

        def run(x_hbm):
            @pl.loop(0, NCH)
            def _(j):
                pltpu.async_copy(x_hbm.at[srcv.at[j]], rows, sem).wait()
                pltpu.sync_copy(rows, acc.at[dstv.at[j]], add=True)

        @pl.when(c == 0)
        def _():
            run(x0_hbm)

        @pl.when(c == 1)
        def _():
            run(x1_hbm)

        plsc.subcore_barrier()
        pltpu.async_copy(acc.at[pl.ds(base, RPW)],
                         s_hbm.at[c, pl.ds(base, RPW)], sem).wait()

    return sc_seg(src3, dst3, x0, x1, z)


BLK = 400


def _tc_matmul_body(s_ref, wt_ref, o_ref):
    a = jnp.dot(s_ref[0], wt_ref[:DH, :],
                preferred_element_type=jnp.float32,
                precision=lax.Precision.HIGHEST)
    b = jnp.dot(s_ref[1], wt_ref[DH:, :],
                preferred_element_type=jnp.float32,
                precision=lax.Precision.HIGHEST)
    o_ref[...] = a + b


_tc_matmul = functools.partial(
    pl.pallas_call,
    out_shape=jax.ShapeDtypeStruct((N, D), jnp.float32),
    grid=(N // BLK,),
    in_specs=[
        pl.BlockSpec((2, BLK, DH), lambda i: (0, i, 0)),
        pl.BlockSpec((D, D), lambda i: (0, 0)),
    ],
    out_specs=pl.BlockSpec((BLK, D), lambda i: (i, 0)),
)(_tc_matmul_body)


@jax.jit
def kernel(edge_index, X, W):
    src3 = edge_index[0].reshape(NS, NCH, C)
    dst3 = edge_index[1].reshape(NS, NCH, C)
    x0 = X[:, :DH]
    x1 = X[:, DH:]
    z = jnp.zeros((NP, DH), jnp.float32)
    s2 = _sc_segment_sum(src3, dst3, x0, x1, z)
    return _tc_matmul(s2, W.T)
